# Initial kernel scaffold; baseline (speedup 1.0000x reference)
import jax
import jax.numpy as jnp
from jax.experimental import pallas as pl


def kernel(x, chan_ind, spat_ind, embed_channel, embed_spatial):
    raise NotImplementedError("write your pallas kernel here")


# speedup vs baseline: 1.6059x; 1.6059x over previous
import functools

import jax
import jax.numpy as jnp
from jax import lax
from jax.experimental import pallas as pl
from jax.experimental.pallas import tpu as pltpu
from jax.experimental.pallas import tpu_sc as plsc

_LANES = 16


def _make_sc_kernel(N, HID, D, NC, NS, C):
    NW = NC * NS
    rows_pw = N // NW
    n_chunks = rows_pw // C
    mesh = plsc.VectorSubcoreMesh(core_axis_name="c", subcore_axis_name="s")

    @functools.partial(
        pl.kernel,
        mesh=mesh,
        out_type=jax.ShapeDtypeStruct((N, HID), jnp.float32),
        scratch_types=[
            pltpu.VMEM((4, rows_pw), jnp.int32),
            pltpu.VMEM((C, HID), jnp.float32),
            pltpu.VMEM((4, C, D), jnp.float32),
            pltpu.SemaphoreType.DMA,
        ],
    )
    def k(x_hbm, idx_hbm, tc_hbm, ts_hbm, out_hbm, idx_v, x_v, e_v, sem):
        wid = lax.axis_index("s") * NC + lax.axis_index("c")
        base = wid * rows_pw
        pltpu.sync_copy(idx_hbm.at[:, pl.ds(base, rows_pw)], idx_v)

        def chunk(kk, _):
            r0 = base + kk * C
            c0 = kk * C
            cx = pltpu.async_copy(x_hbm.at[pl.ds(r0, C), :], x_v, sem)
            g0 = pltpu.async_copy(tc_hbm.at[idx_v.at[0, pl.ds(c0, C)]], e_v.at[0], sem)
            g1 = pltpu.async_copy(tc_hbm.at[idx_v.at[1, pl.ds(c0, C)]], e_v.at[1], sem)
            g2 = pltpu.async_copy(ts_hbm.at[idx_v.at[2, pl.ds(c0, C)]], e_v.at[2], sem)
            g3 = pltpu.async_copy(ts_hbm.at[idx_v.at[3, pl.ds(c0, C)]], e_v.at[3], sem)
            cx.wait()
            g0.wait()
            g1.wait()
            g2.wait()
            g3.wait()

            def row(c, carry):
                for j in range(4):
                    for t in range(D // _LANES):
                        plsc.addupdate(
                            x_v.at[c, pl.ds(j * D + t * _LANES, _LANES)],
                            e_v[j, c, pl.ds(t * _LANES, _LANES)],
                        )
                return carry

            lax.fori_loop(0, C, row, 0, unroll=False)
            pltpu.sync_copy(x_v, out_hbm.at[pl.ds(r0, C), :])
            return _

        lax.fori_loop(0, n_chunks, chunk, 0, unroll=False)

    return k


def kernel(x, chan_ind, spat_ind, embed_channel, embed_spatial):
    N, HID = x.shape
    D = embed_channel.shape[1]
    idx_all = jnp.concatenate(
        [chan_ind.astype(jnp.int32), spat_ind.astype(jnp.int32)], axis=1
    ).T
    info = plsc.get_sparse_core_info()
    k = _make_sc_kernel(N, HID, D, info.num_cores, info.num_subcores, 32)
    return k(x, idx_all, embed_channel, embed_spatial)


# speedup vs baseline: 2.4175x; 1.5054x over previous
import functools

import jax
import jax.numpy as jnp
from jax import lax
from jax.experimental import pallas as pl
from jax.experimental.pallas import tpu as pltpu
from jax.experimental.pallas import tpu_sc as plsc

_LANES = 16


def _make_sc_kernel(N, HID, D, NC, NS, C):
    NW = NC * NS
    rows_pw = N // NW
    n_chunks = rows_pw // C
    mesh = plsc.VectorSubcoreMesh(core_axis_name="c", subcore_axis_name="s")

    @functools.partial(
        pl.kernel,
        mesh=mesh,
        out_type=jax.ShapeDtypeStruct((N, HID), jnp.float32),
        scratch_types=[
            pltpu.VMEM((4, rows_pw), jnp.int32),
            pltpu.VMEM((4, C, HID), jnp.float32),
            pltpu.VMEM((2, 4, C, D), jnp.float32),
            pltpu.SemaphoreType.DMA,
            pltpu.SemaphoreType.DMA,
            pltpu.SemaphoreType.DMA,
            pltpu.SemaphoreType.DMA,
        ],
    )
    def k(x_hbm, idx_hbm, tc_hbm, ts_hbm, out_hbm, idx_v, x_v, e_v, si0, si1, so0, so1):
        s_in = (si0, si1)
        s_out = (so0, so1)
        wid = lax.axis_index("s") * NC + lax.axis_index("c")
        base = wid * rows_pw
        pltpu.sync_copy(idx_hbm.at[:, pl.ds(base, rows_pw)], idx_v)

        def in_copies(g, xs, es):
            r0 = base + g * C
            c0 = g * C
            sem = s_in[es]
            return (
                pltpu.make_async_copy(x_hbm.at[pl.ds(r0, C), :], x_v.at[xs], sem),
                pltpu.make_async_copy(tc_hbm.at[idx_v.at[0, pl.ds(c0, C)]], e_v.at[es, 0], sem),
                pltpu.make_async_copy(tc_hbm.at[idx_v.at[1, pl.ds(c0, C)]], e_v.at[es, 1], sem),
                pltpu.make_async_copy(ts_hbm.at[idx_v.at[2, pl.ds(c0, C)]], e_v.at[es, 2], sem),
                pltpu.make_async_copy(ts_hbm.at[idx_v.at[3, pl.ds(c0, C)]], e_v.at[es, 3], sem),
            )

        def out_copy(g, xs, es):
            r0 = base + g * C
            return pltpu.make_async_copy(
                x_v.at[xs], out_hbm.at[pl.ds(r0, C), :], s_out[es]
            )

        def add_chunk(xs, es):
            def row(c, carry):
                for j in range(4):
                    for t in range(D // _LANES):
                        plsc.addupdate(
                            x_v.at[xs, c, pl.ds(j * D + t * _LANES, _LANES)],
                            e_v[es, j, c, pl.ds(t * _LANES, _LANES)],
                        )
                return carry

            lax.fori_loop(0, C, row, 0, unroll=False)

        def super_chunk(g2, carry):
            for u in range(4):
                g = g2 * 4 + u
                b = u % 2

                @pl.when(g < n_chunks - 1)
                def _fire_next():
                    for d in in_copies(g + 1, (u + 1) % 4, 1 - b):
                        d.start()

                for d in in_copies(g, u, b):
                    d.wait()
                add_chunk(u, b)

                @pl.when(g >= 1)
                def _drain_prev_out():
                    out_copy(g - 1, (u + 3) % 4, 1 - b).wait()

                out_copy(g, u, b).start()
            return carry

        for d in in_copies(0, 0, 0):
            d.start()
        lax.fori_loop(0, n_chunks // 4, super_chunk, 0, unroll=False)
        out_copy(n_chunks - 1, 3, 1).wait()

    return k


def kernel(x, chan_ind, spat_ind, embed_channel, embed_spatial):
    N, HID = x.shape
    D = embed_channel.shape[1]
    idx_all = jnp.concatenate(
        [chan_ind.astype(jnp.int32), spat_ind.astype(jnp.int32)], axis=1
    ).T
    info = plsc.get_sparse_core_info()
    k = _make_sc_kernel(N, HID, D, info.num_cores, info.num_subcores, 16)
    return k(x, idx_all, embed_channel, embed_spatial)
